# baseline (speedup 1.0000x reference)
import functools

import jax
import jax.numpy as jnp
from jax import lax
from jax.experimental import pallas as pl
from jax.experimental.pallas import tpu as pltpu
from jax.experimental.pallas import tpu_sc as plsc

RES = 128
NUM_FRAMES = 8
NUM_BATCHES = 2
N = 2097152
GRID_N = NUM_BATCHES * NUM_FRAMES * RES * RES * RES
GRID_WORDS = GRID_N // 4

NW = 32
PER_W = N // NW
CHUNK = 4096
GROUPS = CHUNK // 16
NCHUNKS = PER_W // CHUNK

_PACK_ROWS = GRID_N // 512
_PACK_BLK = 2048


def _pack_body(src_ref, dst_ref):
    w = src_ref[:, 0, :].astype(jnp.int32)
    for k in range(1, 4):
        w = w | (src_ref[:, k, :].astype(jnp.int32) << (8 * k))
    dst_ref[...] = w


_grid_pack = pl.pallas_call(
    _pack_body,
    out_shape=jax.ShapeDtypeStruct((_PACK_ROWS, 128), jnp.int32),
    grid=(_PACK_ROWS // _PACK_BLK,),
    in_specs=[pl.BlockSpec((_PACK_BLK, 4, 128), lambda i: (i, 0, 0))],
    out_specs=pl.BlockSpec((_PACK_BLK, 128), lambda i: (i, 0)),
)

def _tobool_body(src_ref, dst_ref):
    dst_ref[...] = src_ref[...] != 0


_to_bool = pl.pallas_call(
    _tobool_body,
    out_shape=jax.ShapeDtypeStruct((N // 128, 128), jnp.bool_),
    grid=(4,),
    in_specs=[pl.BlockSpec((N // 512, 128), lambda i: (i, 0))],
    out_specs=pl.BlockSpec((N // 512, 128), lambda i: (i, 0)),
)

_mesh = plsc.VectorSubcoreMesh(core_axis_name="c", subcore_axis_name="s",
                               num_cores=2, num_subcores=16)


@functools.partial(
    pl.kernel,
    out_type=jax.ShapeDtypeStruct((N,), jnp.int32),
    mesh=_mesh,
    compiler_params=pltpu.CompilerParams(needs_layout_passes=False),
    scratch_types=[
        pltpu.VMEM((CHUNK * 3,), jnp.float32),
        pltpu.VMEM((CHUNK,), jnp.int32),
        pltpu.VMEM((CHUNK,), jnp.float32),
        pltpu.VMEM((CHUNK,), jnp.int32),
        pltpu.VMEM((CHUNK,), jnp.int32),
        pltpu.VMEM((CHUNK,), jnp.int32),
        pltpu.VMEM((CHUNK,), jnp.int32),
        pltpu.VMEM((16,), jnp.float32),
        pltpu.SemaphoreType.DMA,
    ],
)
def _occ_query(pts_hbm, bidx_hbm, ts_hbm, gridw_hbm, kf_hbm, out_hbm,
               pts_v, bidx_v, ts_v, widx_v, shift_v, words_v, out_v,
               kf_v, sem):
    wid = lax.axis_index("s") * 2 + lax.axis_index("c")
    pltpu.sync_copy(kf_hbm, kf_v)

    lanes = lax.iota(jnp.int32, 16)
    zeros = jnp.zeros((16,), jnp.int32)
    kfvec = kf_v[...]
    kfs = [kfvec[j] for j in range(NUM_FRAMES)]

    def chunk_body(c, carry):
        base = wid * PER_W + c * CHUNK
        pltpu.sync_copy(pts_hbm.at[pl.ds(base * 3, CHUNK * 3)], pts_v)
        pltpu.sync_copy(bidx_hbm.at[pl.ds(base, CHUNK)], bidx_v)
        pltpu.sync_copy(ts_hbm.at[pl.ds(base, CHUNK)], ts_v)

        def group_body(g, carry2):
            o = pl.multiple_of(g * 16, 16)
            rows3 = (o + lanes) * 3
            x = plsc.load_gather(pts_v, [rows3])
            y = plsc.load_gather(pts_v, [rows3 + 1])
            z = plsc.load_gather(pts_v, [rows3 + 2])
            t = ts_v[pl.ds(o, 16)]
            bi = bidx_v[pl.ds(o, 16)]

            def cell(v):
                q = ((v / 2.0 + 0.5) * 128.0).astype(jnp.int32)
                return jnp.clip(q, 0, RES - 1)

            gx, gy, gz = cell(x), cell(y), cell(z)

            cnt = zeros
            for j in range(NUM_FRAMES):
                cnt = cnt + jnp.where(kfs[j] < t, 1, 0)
            idx = jnp.clip(cnt, 1, NUM_FRAMES - 1)
            left = jnp.full((16,), kfs[NUM_FRAMES - 2])
            right = jnp.full((16,), kfs[NUM_FRAMES - 1])
            for j in range(NUM_FRAMES - 2, 0, -1):
                m = idx == j
                left = jnp.where(m, kfs[j - 1], left)
                right = jnp.where(m, kfs[j], right)
            fidx = jnp.where(jnp.abs(t - left) <= jnp.abs(right - t),
                             idx - 1, idx)

            flat = ((bi * NUM_FRAMES + fidx) * (RES * RES * RES)
                    + gx * (RES * RES) + gy * RES + gz)
            widx_v[pl.ds(o, 16)] = (
                (lax.shift_right_logical(flat, 2) & -128) | (flat & 127))
            shift_v[pl.ds(o, 16)] = (lax.shift_right_logical(flat, 7) & 3) * 8
            return carry2

        lax.fori_loop(0, GROUPS, group_body, 0)

        pltpu.async_copy(gridw_hbm.at[widx_v], words_v, sem).wait()

        def bit_body(g, carry2):
            o = pl.multiple_of(g * 16, 16)
            w = words_v[pl.ds(o, 16)]
            s = shift_v[pl.ds(o, 16)]
            out_v[pl.ds(o, 16)] = lax.shift_right_logical(w, s) & 1
            return carry2

        lax.fori_loop(0, GROUPS, bit_body, 0)
        pltpu.sync_copy(out_v, out_hbm.at[pl.ds(base, CHUNK)])
        return carry

    lax.fori_loop(0, NCHUNKS, chunk_body, 0)


def kernel(pts, bidx, ts, flat_occ_grid, ts_keyframes):
    gw = _grid_pack(flat_occ_grid.reshape(_PACK_ROWS, 4, 128))
    kf16 = jnp.pad(ts_keyframes, (0, 16 - NUM_FRAMES))
    occ_w = _occ_query(pts.reshape(-1), bidx, ts, gw.reshape(-1), kf16)
    return _to_bool(occ_w.reshape(N // 128, 128)).reshape(N)


# speedup vs baseline: 1.0002x; 1.0002x over previous
import functools

import jax
import jax.numpy as jnp
from jax import lax
from jax.experimental import pallas as pl
from jax.experimental.pallas import tpu as pltpu
from jax.experimental.pallas import tpu_sc as plsc

RES = 128
NUM_FRAMES = 8
NUM_BATCHES = 2
N = 2097152
GRID_N = NUM_BATCHES * NUM_FRAMES * RES * RES * RES
GRID_WORDS = GRID_N // 4

NW = 32
PER_W = N // NW
CHUNK = 4096
GROUPS = CHUNK // 16
NCHUNKS = PER_W // CHUNK

_PACK_ROWS = GRID_N // 512
_PACK_BLK = 2048


def _pack_body(src_ref, dst_ref):
    w = src_ref[:, 0, :].astype(jnp.int32)
    for k in range(1, 4):
        w = w | (src_ref[:, k, :].astype(jnp.int32) << (8 * k))
    dst_ref[...] = w.reshape(-1)


_grid_pack = pl.pallas_call(
    _pack_body,
    out_shape=jax.ShapeDtypeStruct((GRID_WORDS,), jnp.int32),
    grid=(_PACK_ROWS // _PACK_BLK,),
    in_specs=[pl.BlockSpec((_PACK_BLK, 4, 128), lambda i: (i, 0, 0))],
    out_specs=pl.BlockSpec((_PACK_BLK * 128,), lambda i: (i,)),
)

def _tobool_body(src_ref, dst_ref):
    dst_ref[...] = src_ref[...] != 0


_to_bool = pl.pallas_call(
    _tobool_body,
    out_shape=jax.ShapeDtypeStruct((N,), jnp.bool_),
    grid=(4,),
    in_specs=[pl.BlockSpec((N // 4,), lambda i: (i,))],
    out_specs=pl.BlockSpec((N // 4,), lambda i: (i,)),
)

_mesh = plsc.VectorSubcoreMesh(core_axis_name="c", subcore_axis_name="s",
                               num_cores=2, num_subcores=16)


@functools.partial(
    pl.kernel,
    out_type=jax.ShapeDtypeStruct((N,), jnp.int32),
    mesh=_mesh,
    compiler_params=pltpu.CompilerParams(needs_layout_passes=False),
    scratch_types=[
        pltpu.VMEM((CHUNK * 3,), jnp.float32),
        pltpu.VMEM((CHUNK,), jnp.int32),
        pltpu.VMEM((CHUNK,), jnp.float32),
        pltpu.VMEM((CHUNK,), jnp.int32),
        pltpu.VMEM((CHUNK,), jnp.int32),
        pltpu.VMEM((CHUNK,), jnp.int32),
        pltpu.VMEM((CHUNK,), jnp.int32),
        pltpu.VMEM((16,), jnp.float32),
        pltpu.SemaphoreType.DMA,
    ],
)
def _occ_query(pts_hbm, bidx_hbm, ts_hbm, gridw_hbm, kf_hbm, out_hbm,
               pts_v, bidx_v, ts_v, widx_v, shift_v, words_v, out_v,
               kf_v, sem):
    wid = lax.axis_index("s") * 2 + lax.axis_index("c")
    pltpu.sync_copy(kf_hbm, kf_v)

    lanes = lax.iota(jnp.int32, 16)
    zeros = jnp.zeros((16,), jnp.int32)
    kfvec = kf_v[...]
    kfs = [kfvec[j] for j in range(NUM_FRAMES)]

    def chunk_body(c, carry):
        base = wid * PER_W + c * CHUNK
        pltpu.sync_copy(pts_hbm.at[pl.ds(base * 3, CHUNK * 3)], pts_v)
        pltpu.sync_copy(bidx_hbm.at[pl.ds(base, CHUNK)], bidx_v)
        pltpu.sync_copy(ts_hbm.at[pl.ds(base, CHUNK)], ts_v)

        def group_body(g, carry2):
            o = pl.multiple_of(g * 16, 16)
            rows3 = (o + lanes) * 3
            x = plsc.load_gather(pts_v, [rows3])
            y = plsc.load_gather(pts_v, [rows3 + 1])
            z = plsc.load_gather(pts_v, [rows3 + 2])
            t = ts_v[pl.ds(o, 16)]
            bi = bidx_v[pl.ds(o, 16)]

            def cell(v):
                q = ((v / 2.0 + 0.5) * 128.0).astype(jnp.int32)
                return jnp.clip(q, 0, RES - 1)

            gx, gy, gz = cell(x), cell(y), cell(z)

            cnt = zeros
            for j in range(NUM_FRAMES):
                cnt = cnt + jnp.where(kfs[j] < t, 1, 0)
            idx = jnp.clip(cnt, 1, NUM_FRAMES - 1)
            left = jnp.full((16,), kfs[NUM_FRAMES - 2])
            right = jnp.full((16,), kfs[NUM_FRAMES - 1])
            for j in range(NUM_FRAMES - 2, 0, -1):
                m = idx == j
                left = jnp.where(m, kfs[j - 1], left)
                right = jnp.where(m, kfs[j], right)
            fidx = jnp.where(jnp.abs(t - left) <= jnp.abs(right - t),
                             idx - 1, idx)

            flat = ((bi * NUM_FRAMES + fidx) * (RES * RES * RES)
                    + gx * (RES * RES) + gy * RES + gz)
            widx_v[pl.ds(o, 16)] = (
                (lax.shift_right_logical(flat, 2) & -128) | (flat & 127))
            shift_v[pl.ds(o, 16)] = (lax.shift_right_logical(flat, 7) & 3) * 8
            return carry2

        lax.fori_loop(0, GROUPS, group_body, 0)

        pltpu.async_copy(gridw_hbm.at[widx_v], words_v, sem).wait()

        def bit_body(g, carry2):
            o = pl.multiple_of(g * 16, 16)
            w = words_v[pl.ds(o, 16)]
            s = shift_v[pl.ds(o, 16)]
            out_v[pl.ds(o, 16)] = lax.shift_right_logical(w, s) & 1
            return carry2

        lax.fori_loop(0, GROUPS, bit_body, 0)
        pltpu.sync_copy(out_v, out_hbm.at[pl.ds(base, CHUNK)])
        return carry

    lax.fori_loop(0, NCHUNKS, chunk_body, 0)


def kernel(pts, bidx, ts, flat_occ_grid, ts_keyframes):
    gw = _grid_pack(flat_occ_grid.reshape(_PACK_ROWS, 4, 128))
    kf16 = jnp.pad(ts_keyframes, (0, 16 - NUM_FRAMES))
    occ_w = _occ_query(pts.reshape(-1), bidx, ts, gw, kf16)
    return _to_bool(occ_w)


# speedup vs baseline: 1.0006x; 1.0003x over previous
import functools

import jax
import jax.numpy as jnp
from jax import lax
from jax.experimental import pallas as pl
from jax.experimental.pallas import tpu as pltpu
from jax.experimental.pallas import tpu_sc as plsc

RES = 128
NUM_FRAMES = 8
NUM_BATCHES = 2
N = 2097152
GRID_N = NUM_BATCHES * NUM_FRAMES * RES * RES * RES
GRID_WORDS = GRID_N // 4

NW = 32
PER_W = N // NW
CHUNK = 4096
GROUPS = CHUNK // 16
NCHUNKS = PER_W // CHUNK

_PACK_ROWS = GRID_N // 512
_PACK_BLK = 2048


def _pack_body(src_ref, dst_ref):
    for x0 in range(RES // 4):
        w = src_ref[0, 4 * x0, :, :].astype(jnp.int32)
        for k in range(1, 4):
            w = w | (src_ref[0, 4 * x0 + k, :, :].astype(jnp.int32)
                     << (8 * k))
        dst_ref[pl.ds(x0 * RES * RES, RES * RES)] = w.reshape(-1)


_grid_pack = pl.pallas_call(
    _pack_body,
    out_shape=jax.ShapeDtypeStruct((GRID_WORDS,), jnp.int32),
    grid=(NUM_BATCHES * NUM_FRAMES,),
    in_specs=[pl.BlockSpec((1, RES, RES, RES), lambda i: (i, 0, 0, 0))],
    out_specs=pl.BlockSpec((RES * RES * RES // 4,), lambda i: (i,)),
)

def _tobool_body(src_ref, dst_ref):
    dst_ref[...] = src_ref[...] != 0


_to_bool = pl.pallas_call(
    _tobool_body,
    out_shape=jax.ShapeDtypeStruct((N,), jnp.bool_),
    grid=(4,),
    in_specs=[pl.BlockSpec((N // 4,), lambda i: (i,))],
    out_specs=pl.BlockSpec((N // 4,), lambda i: (i,)),
)

_mesh = plsc.VectorSubcoreMesh(core_axis_name="c", subcore_axis_name="s",
                               num_cores=2, num_subcores=16)


@functools.partial(
    pl.kernel,
    out_type=jax.ShapeDtypeStruct((N,), jnp.int32),
    mesh=_mesh,
    compiler_params=pltpu.CompilerParams(needs_layout_passes=False),
    scratch_types=[
        pltpu.VMEM((CHUNK * 3,), jnp.float32),
        pltpu.VMEM((CHUNK,), jnp.int32),
        pltpu.VMEM((CHUNK,), jnp.float32),
        pltpu.VMEM((CHUNK,), jnp.int32),
        pltpu.VMEM((CHUNK,), jnp.int32),
        pltpu.VMEM((CHUNK,), jnp.int32),
        pltpu.VMEM((CHUNK,), jnp.int32),
        pltpu.VMEM((16,), jnp.float32),
        pltpu.SemaphoreType.DMA,
    ],
)
def _occ_query(pts_hbm, bidx_hbm, ts_hbm, gridw_hbm, kf_hbm, out_hbm,
               pts_v, bidx_v, ts_v, widx_v, shift_v, words_v, out_v,
               kf_v, sem):
    wid = lax.axis_index("s") * 2 + lax.axis_index("c")
    pltpu.sync_copy(kf_hbm, kf_v)

    lanes = lax.iota(jnp.int32, 16)
    zeros = jnp.zeros((16,), jnp.int32)
    kfvec = kf_v[...]
    kfs = [kfvec[j] for j in range(NUM_FRAMES)]

    def chunk_body(c, carry):
        base = wid * PER_W + c * CHUNK
        pltpu.sync_copy(pts_hbm.at[pl.ds(base * 3, CHUNK * 3)], pts_v)
        pltpu.sync_copy(bidx_hbm.at[pl.ds(base, CHUNK)], bidx_v)
        pltpu.sync_copy(ts_hbm.at[pl.ds(base, CHUNK)], ts_v)

        def group_body(g, carry2):
            o = pl.multiple_of(g * 16, 16)
            rows3 = (o + lanes) * 3
            x = plsc.load_gather(pts_v, [rows3])
            y = plsc.load_gather(pts_v, [rows3 + 1])
            z = plsc.load_gather(pts_v, [rows3 + 2])
            t = ts_v[pl.ds(o, 16)]
            bi = bidx_v[pl.ds(o, 16)]

            def cell(v):
                q = ((v / 2.0 + 0.5) * 128.0).astype(jnp.int32)
                return jnp.clip(q, 0, RES - 1)

            gx, gy, gz = cell(x), cell(y), cell(z)

            cnt = zeros
            for j in range(NUM_FRAMES):
                cnt = cnt + jnp.where(kfs[j] < t, 1, 0)
            idx = jnp.clip(cnt, 1, NUM_FRAMES - 1)
            left = jnp.full((16,), kfs[NUM_FRAMES - 2])
            right = jnp.full((16,), kfs[NUM_FRAMES - 1])
            for j in range(NUM_FRAMES - 2, 0, -1):
                m = idx == j
                left = jnp.where(m, kfs[j - 1], left)
                right = jnp.where(m, kfs[j], right)
            fidx = jnp.where(jnp.abs(t - left) <= jnp.abs(right - t),
                             idx - 1, idx)

            flat = ((bi * NUM_FRAMES + fidx) * (RES * RES * RES)
                    + gx * (RES * RES) + gy * RES + gz)
            widx_v[pl.ds(o, 16)] = (
                (lax.shift_right_logical(flat, 2) & -16384) | (flat & 16383))
            shift_v[pl.ds(o, 16)] = (
                lax.shift_right_logical(flat, 14) & 3) * 8
            return carry2

        lax.fori_loop(0, GROUPS, group_body, 0)

        pltpu.async_copy(gridw_hbm.at[widx_v], words_v, sem).wait()

        def bit_body(g, carry2):
            o = pl.multiple_of(g * 16, 16)
            w = words_v[pl.ds(o, 16)]
            s = shift_v[pl.ds(o, 16)]
            out_v[pl.ds(o, 16)] = lax.shift_right_logical(w, s) & 1
            return carry2

        lax.fori_loop(0, GROUPS, bit_body, 0)
        pltpu.sync_copy(out_v, out_hbm.at[pl.ds(base, CHUNK)])
        return carry

    lax.fori_loop(0, NCHUNKS, chunk_body, 0)


def kernel(pts, bidx, ts, flat_occ_grid, ts_keyframes):
    gw = _grid_pack(flat_occ_grid)
    kf16 = jnp.pad(ts_keyframes, (0, 16 - NUM_FRAMES))
    occ_w = _occ_query(pts.reshape(-1), bidx, ts, gw, kf16)
    return _to_bool(occ_w)


# speedup vs baseline: 12.1522x; 12.1454x over previous
import functools

import jax
import jax.numpy as jnp
from jax import lax
from jax.experimental import pallas as pl
from jax.experimental.pallas import tpu as pltpu
from jax.experimental.pallas import tpu_sc as plsc

RES = 128
NUM_FRAMES = 8
NUM_BATCHES = 2
N = 2097152
GRID_N = NUM_BATCHES * NUM_FRAMES * RES * RES * RES

NW = 32
PER_W = N // NW
CHUNK = 8192
GROUPS = CHUNK // 16
NCHUNKS = PER_W // CHUNK

def _tobool_body(src_ref, dst_ref):
    dst_ref[...] = src_ref[...] != 0


_to_bool = pl.pallas_call(
    _tobool_body,
    out_shape=jax.ShapeDtypeStruct((N,), jnp.bool_),
    grid=(4,),
    in_specs=[pl.BlockSpec((N // 4,), lambda i: (i,))],
    out_specs=pl.BlockSpec((N // 4,), lambda i: (i,)),
)

_mesh = plsc.VectorSubcoreMesh(core_axis_name="c", subcore_axis_name="s",
                               num_cores=2, num_subcores=16)


@functools.partial(
    pl.kernel,
    out_type=jax.ShapeDtypeStruct((N,), jnp.int32),
    mesh=_mesh,
    compiler_params=pltpu.CompilerParams(needs_layout_passes=False),
    scratch_types=[
        pltpu.VMEM((CHUNK,), jnp.float32),
        pltpu.VMEM((CHUNK,), jnp.float32),
        pltpu.VMEM((CHUNK,), jnp.float32),
        pltpu.VMEM((CHUNK,), jnp.int32),
        pltpu.VMEM((CHUNK,), jnp.float32),
        pltpu.VMEM((CHUNK,), jnp.int32),
        pltpu.VMEM((CHUNK,), jnp.int32),
        pltpu.VMEM((CHUNK,), jnp.int32),
        pltpu.VMEM((CHUNK,), jnp.int32),
        pltpu.VMEM((16,), jnp.float32),
        pltpu.SemaphoreType.DMA,
        pltpu.SemaphoreType.DMA,
    ],
)
def _occ_query(x_hbm, y_hbm, z_hbm, bidx_hbm, ts_hbm, gridw_hbm, kf_hbm,
               out_hbm, x_v, y_v, z_v, bidx_v, ts_v, widx_a, widx_b,
               words_a, words_b, kf_v, sem_a, sem_b):
    wid = lax.axis_index("s") * 2 + lax.axis_index("c")
    pltpu.sync_copy(kf_hbm, kf_v)

    zeros = jnp.zeros((16,), jnp.int32)
    kfvec = kf_v[...]
    kfs = [kfvec[j] for j in range(NUM_FRAMES)]

    def compute_idx(c, widx_v):
        base = wid * PER_W + c * CHUNK
        pltpu.sync_copy(x_hbm.at[pl.ds(base, CHUNK)], x_v)
        pltpu.sync_copy(y_hbm.at[pl.ds(base, CHUNK)], y_v)
        pltpu.sync_copy(z_hbm.at[pl.ds(base, CHUNK)], z_v)
        pltpu.sync_copy(bidx_hbm.at[pl.ds(base, CHUNK)], bidx_v)
        pltpu.sync_copy(ts_hbm.at[pl.ds(base, CHUNK)], ts_v)

        def group_body(g, carry2):
            o = pl.multiple_of(g * 16, 16)
            x = x_v[pl.ds(o, 16)]
            y = y_v[pl.ds(o, 16)]
            z = z_v[pl.ds(o, 16)]
            t = ts_v[pl.ds(o, 16)]
            bi = bidx_v[pl.ds(o, 16)]

            def cell(v):
                q = ((v / 2.0 + 0.5) * 128.0).astype(jnp.int32)
                return jnp.clip(q, 0, RES - 1)

            gx, gy, gz = cell(x), cell(y), cell(z)

            cnt = zeros
            for j in range(NUM_FRAMES):
                cnt = cnt + jnp.where(kfs[j] < t, 1, 0)
            idx = jnp.clip(cnt, 1, NUM_FRAMES - 1)
            left = jnp.full((16,), kfs[NUM_FRAMES - 2])
            right = jnp.full((16,), kfs[NUM_FRAMES - 1])
            for j in range(NUM_FRAMES - 2, 0, -1):
                m = idx == j
                left = jnp.where(m, kfs[j - 1], left)
                right = jnp.where(m, kfs[j], right)
            fidx = jnp.where(jnp.abs(t - left) <= jnp.abs(right - t),
                             idx - 1, idx)

            flat = ((bi * NUM_FRAMES + fidx) * (RES * RES * RES)
                    + gx * (RES * RES) + gy * RES + gz)
            widx_v[pl.ds(o, 16)] = flat
            return carry2

        lax.fori_loop(0, GROUPS, group_body, 0)

    def write_out(c, words_v):
        base = wid * PER_W + c * CHUNK
        pltpu.sync_copy(words_v, out_hbm.at[pl.ds(base, CHUNK)])

    def gather(widx_v, words_v, sem):
        return pltpu.async_copy(gridw_hbm.at[widx_v], words_v, sem)

    compute_idx(0, widx_a)

    def chunk_pair(i, carry):
        ca = i * 2
        ha = gather(widx_a, words_a, sem_a)
        compute_idx(ca + 1, widx_b)
        hb = gather(widx_b, words_b, sem_b)
        ha.wait()
        write_out(ca, words_a)

        @pl.when(ca + 2 < NCHUNKS)
        def _():
            compute_idx(ca + 2, widx_a)

        hb.wait()
        write_out(ca + 1, words_b)
        return carry

    lax.fori_loop(0, NCHUNKS // 2, chunk_pair, 0)


def kernel(pts, bidx, ts, flat_occ_grid, ts_keyframes):
    gw = flat_occ_grid.reshape(-1).astype(jnp.int32)
    kf16 = jnp.pad(ts_keyframes, (0, 16 - NUM_FRAMES))
    occ_w = _occ_query(pts[:, 0], pts[:, 1], pts[:, 2], bidx, ts, gw, kf16)
    return _to_bool(occ_w)
